# baseline (speedup 1.0000x reference)
import functools

import jax
import jax.numpy as jnp
from jax import lax
from jax.experimental import pallas as pl
from jax.experimental.pallas import tpu as pltpu
from jax.experimental.pallas import tpu_sc as plsc

_NC, _NS = 2, 16
_NW = _NC * _NS
_CHUNK = 1024


@functools.lru_cache(maxsize=None)
def _build(B, V, D):
    assert B % (_NW * _CHUNK) == 0
    b_per_w = B // _NW
    n_steps = b_per_w // _CHUNK
    assert n_steps % 3 == 1 and n_steps >= 4
    mesh = plsc.VectorSubcoreMesh(core_axis_name="c", subcore_axis_name="s")

    @functools.partial(
        pl.kernel,
        out_type=jax.ShapeDtypeStruct((B, D), jnp.float32),
        mesh=mesh,
        compiler_params=pltpu.CompilerParams(use_tc_tiling_on_sc=False),
        scratch_types=[
            pltpu.VMEM((3, _CHUNK), jnp.int32),
            pltpu.VMEM((3, _CHUNK, D), jnp.float32),
            pltpu.SemaphoreType.DMA,
            pltpu.SemaphoreType.DMA,
            pltpu.SemaphoreType.DMA,
            pltpu.SemaphoreType.DMA,
            pltpu.SemaphoreType.DMA,
            pltpu.SemaphoreType.DMA,
            pltpu.SemaphoreType.DMA,
            pltpu.SemaphoreType.DMA,
            pltpu.SemaphoreType.DMA,
        ],
    )
    def gather_kernel(idx_hbm, table_hbm, out_hbm, idx_v, rows_v,
                      si0, si1, si2, sg0, sg1, sg2, so0, so1, so2):
        sem_idx = [si0, si1, si2]
        sem_g = [sg0, sg1, sg2]
        sem_o = [so0, so1, so2]
        wid = lax.axis_index("s") * _NC + lax.axis_index("c")
        row0 = wid * n_steps
        out_base = wid * b_per_w

        def idx_load(g, s):
            return pltpu.make_async_copy(
                idx_hbm.at[row0 + g], idx_v.at[s], sem_idx[s])

        def gather(s):
            return pltpu.make_async_copy(
                table_hbm.at[idx_v.at[s]], rows_v.at[s], sem_g[s])

        def out_write(g, s):
            return pltpu.make_async_copy(
                rows_v.at[s],
                out_hbm.at[pl.ds(out_base + g * _CHUNK, _CHUNK)],
                sem_o[s])

        idx_load(0, 0).start()
        idx_load(1, 1).start()
        idx_load(2, 2).start()

        def outer(G, carry):
            for s in (0, 1, 2):
                g = G * 3 + s
                idx_load(g, s).wait()
                @pl.when(G > 0)
                def _():
                    out_write(g - 3, s).wait()
                gather(s).start()
                prev = (s - 1) % 3
                prev_ready = (G > 0) if s == 0 else True
                @pl.when(prev_ready)
                def _():
                    gather(prev).wait()
                    out_write(g - 1, prev).start()
                load_ok = prev_ready if s != 2 else (G < (n_steps - 1) // 3 - 1)
                @pl.when(load_ok)
                def _():
                    idx_load(g + 2, prev).start()
            return carry

        lax.fori_loop(0, (n_steps - 1) // 3, outer, 0, unroll=False)

        last = n_steps - 1
        idx_load(last, 0).wait()
        out_write(last - 3, 0).wait()
        gather(0).start()
        gather(2).wait()
        out_write(last - 1, 2).start()
        gather(0).wait()
        out_write(last, 0).start()
        out_write(last - 2, 1).wait()
        out_write(last - 1, 2).wait()
        out_write(last, 0).wait()

    return gather_kernel


def kernel(value, table):
    bsz, seq = value.shape
    V, D = table.shape
    B = bsz * seq
    idx2d = value.astype(jnp.int32).reshape(B // _CHUNK, _CHUNK)
    out = _build(B, V, D)(idx2d, table)
    return out.reshape(bsz, seq, D)
